# baseline (speedup 1.0000x reference)
import functools

import jax
import jax.numpy as jnp
from jax import lax
from jax.experimental import pallas as pl
from jax.experimental.pallas import tpu as pltpu
from jax.experimental.pallas import tpu_sc as plsc

B = 16384
D = 64
NC = 2
NS = 16
NW = NC * NS
BPW = B // NW
IDX_CH = BPW // 128

BT = 2048
NB = B // BT

TBLK = 4096

_mesh = plsc.VectorSubcoreMesh(core_axis_name="c", subcore_axis_name="s")


def _pack_body(xt_ref, ident_ref, out_ref):
    x = xt_ref[...]
    ident = ident_ref[...]
    xt = lax.dot_general(x, ident, (((0,), (0,)), ((), ())),
                         preferred_element_type=jnp.float32)
    x3 = xt.reshape(TBLK // 2, 2, D)
    out_ref[...] = jnp.concatenate([x3[:, 0, :], x3[:, 1, :]], axis=1)


def _make_pack(n_rows):
    return pl.pallas_call(
        _pack_body,
        grid=(pl.cdiv(n_rows // 2, TBLK // 2),),
        in_specs=[
            pl.BlockSpec((D, TBLK), lambda i: (0, i)),
            pl.BlockSpec((D, D), lambda i: (0, 0)),
        ],
        out_specs=pl.BlockSpec((TBLK // 2, 128), lambda i: (i, 0)),
        out_shape=jax.ShapeDtypeStruct((n_rows // 2, 128), jnp.float32),
    )


_pack_user = _make_pack(1000000)
_pack_movie = _make_pack(100000)


@functools.partial(
    pl.kernel,
    mesh=_mesh,
    out_type=[
        jax.ShapeDtypeStruct((B, 128), jnp.float32),
        jax.ShapeDtypeStruct((B, 128), jnp.float32),
    ],
    scratch_types=[
        pltpu.VMEM((IDX_CH, 128), jnp.int32),
        pltpu.VMEM((IDX_CH, 128), jnp.int32),
        pltpu.VMEM((BPW, 128), jnp.float32),
        pltpu.SemaphoreType.DMA,
    ],
)
def _sc_gather(uidx_hbm, midx_hbm, up_hbm, mp_hbm, ue_out, me_out,
               uidx_v, midx_v, rows_v, sem):
    wid = lax.axis_index("s") * NC + lax.axis_index("c")
    base = wid * BPW
    pltpu.sync_copy(uidx_hbm.at[wid], uidx_v)
    pltpu.sync_copy(midx_hbm.at[wid], midx_v)
    for idx_v, src, dst in ((uidx_v, up_hbm, ue_out), (midx_v, mp_hbm, me_out)):
        copies = [
            pltpu.async_copy(
                src.at[idx_v.at[j]], rows_v.at[pl.ds(j * 128, 128)], sem)
            for j in range(IDX_CH)
        ]
        for c in copies:
            c.wait()
        pltpu.sync_copy(rows_v, dst.at[pl.ds(base, BPW)])


def _tc_mlp_body(ue_ref, me_ref, up_ref, mp_ref, g_ref, gf_ref, w1_ref,
                 b1_ref, w2_ref, b2_ref, w3_ref, b3_ref, out_ref):
    ue = ue_ref[...]
    me = me_ref[...]
    lane = lax.broadcasted_iota(jnp.int32, (BT, 128), 1)
    upar = lax.bitwise_and(up_ref[...], 1)
    mpar = lax.bitwise_and(mp_ref[...], 1)
    uem = jnp.where((lane >= D) == (upar == 1), ue, 0.0)
    mem = jnp.where((lane >= D) == (mpar == 1), me, 0.0)
    g = g_ref[0]
    ohT = (lax.broadcasted_iota(jnp.int32, (32, BT), 0) == g).astype(jnp.float32)
    w1 = w1_ref[...]
    w1u = jnp.concatenate([w1[0:D, :], w1[0:D, :]], axis=0)
    w1m = jnp.concatenate([w1[D:2 * D, :], w1[D:2 * D, :]], axis=0)
    gcon = jnp.dot(gf_ref[...], w1[2 * D:3 * D, :],
                   preferred_element_type=jnp.float32)
    h1 = jnp.dot(uem, w1u, preferred_element_type=jnp.float32)
    h1 += jnp.dot(mem, w1m, preferred_element_type=jnp.float32)
    h1 += lax.dot_general(ohT, gcon, (((0,), (0,)), ((), ())),
                          preferred_element_type=jnp.float32)
    h1 = jnp.maximum(h1 + b1_ref[...], 0.0)
    h2 = jnp.dot(h1, w2_ref[...], preferred_element_type=jnp.float32)
    h2 = jnp.maximum(h2 + b2_ref[...], 0.0)
    out = jnp.sum(h2 * w3_ref[...], axis=1, keepdims=True) + b3_ref[...]
    out_ref[...] = out


_tc_mlp = pl.pallas_call(
    _tc_mlp_body,
    grid=(NB,),
    in_specs=[
        pl.BlockSpec((BT, 128), lambda i: (i, 0)),
        pl.BlockSpec((BT, 128), lambda i: (i, 0)),
        pl.BlockSpec((BT, 1), lambda i: (i, 0)),
        pl.BlockSpec((BT, 1), lambda i: (i, 0)),
        pl.BlockSpec((1, 1, BT), lambda i: (i, 0, 0)),
        pl.BlockSpec((32, D), lambda i: (0, 0)),
        pl.BlockSpec((3 * D, 128), lambda i: (0, 0)),
        pl.BlockSpec((1, 128), lambda i: (0, 0)),
        pl.BlockSpec((128, D), lambda i: (0, 0)),
        pl.BlockSpec((1, D), lambda i: (0, 0)),
        pl.BlockSpec((1, D), lambda i: (0, 0)),
        pl.BlockSpec((1, 1), lambda i: (0, 0)),
    ],
    out_specs=pl.BlockSpec((BT, 1), lambda i: (i, 0)),
    out_shape=jax.ShapeDtypeStruct((B, 1), jnp.float32),
)


def kernel(user, movie, genres, user_factors, movie_factors, genre_factors,
           W1, b1, W2, b2, W3, b3):
    ident = jnp.eye(D, dtype=jnp.float32)
    up = _pack_user(user_factors.T, ident)
    mp = _pack_movie(movie_factors.T, ident)
    user = user.astype(jnp.int32)
    movie = movie.astype(jnp.int32)
    usup = lax.shift_right_logical(user, 1).reshape(NW, IDX_CH, 128)
    msup = lax.shift_right_logical(movie, 1).reshape(NW, IDX_CH, 128)
    ue, me = _sc_gather(usup, msup, up, mp)
    g3 = genres.astype(jnp.int32).reshape(NB, 1, BT)
    return _tc_mlp(ue, me, user.reshape(B, 1), movie.reshape(B, 1),
                   g3, genre_factors,
                   W1, b1.reshape(1, 128), W2, b2.reshape(1, D),
                   W3.reshape(1, D), b3.reshape(1, 1))


# speedup vs baseline: 1.3370x; 1.3370x over previous
import functools

import jax
import jax.numpy as jnp
from jax import lax
from jax.experimental import pallas as pl
from jax.experimental.pallas import tpu as pltpu
from jax.experimental.pallas import tpu_sc as plsc

B = 16384
D = 64
NC = 2
NS = 16
NW = NC * NS
BPW = B // NW
IDX_CH = BPW // 128

BT = 2048
NB = B // BT

TBLK = 4096

_mesh = plsc.VectorSubcoreMesh(core_axis_name="c", subcore_axis_name="s")


def _pack_body(xt_ref, ident_ref, out_ref):
    x = xt_ref[...]
    ident = ident_ref[...]
    xt = lax.dot_general(x, ident, (((0,), (0,)), ((), ())),
                         preferred_element_type=jnp.float32)
    out_ref[...] = jnp.concatenate(
        [xt[:TBLK // 2, :], xt[TBLK // 2:, :]], axis=1)


def _make_pack(n_rows):
    n_blocks = pl.cdiv(n_rows, TBLK)
    return pl.pallas_call(
        _pack_body,
        grid=(n_blocks,),
        in_specs=[
            pl.BlockSpec((D, TBLK), lambda i: (0, i)),
            pl.BlockSpec((D, D), lambda i: (0, 0)),
        ],
        out_specs=pl.BlockSpec((TBLK // 2, 128), lambda i: (i, 0)),
        out_shape=jax.ShapeDtypeStruct((n_blocks * (TBLK // 2), 128),
                                       jnp.float32),
    )


_pack_user = _make_pack(1000000)
_pack_movie = _make_pack(100000)


@functools.partial(
    pl.kernel,
    mesh=_mesh,
    out_type=[
        jax.ShapeDtypeStruct((B, 128), jnp.float32),
        jax.ShapeDtypeStruct((B, 128), jnp.float32),
    ],
    scratch_types=[
        pltpu.VMEM((IDX_CH, 128), jnp.int32),
        pltpu.VMEM((IDX_CH, 128), jnp.int32),
        pltpu.VMEM((BPW, 128), jnp.float32),
        pltpu.SemaphoreType.DMA,
    ],
)
def _sc_gather(uidx_hbm, midx_hbm, up_hbm, mp_hbm, ue_out, me_out,
               uidx_v, midx_v, rows_v, sem):
    wid = lax.axis_index("s") * NC + lax.axis_index("c")
    base = wid * BPW
    pltpu.sync_copy(uidx_hbm.at[wid], uidx_v)
    pltpu.sync_copy(midx_hbm.at[wid], midx_v)
    for idx_v, src, dst in ((uidx_v, up_hbm, ue_out), (midx_v, mp_hbm, me_out)):
        copies = [
            pltpu.async_copy(
                src.at[idx_v.at[j]], rows_v.at[pl.ds(j * 128, 128)], sem)
            for j in range(IDX_CH)
        ]
        for c in copies:
            c.wait()
        pltpu.sync_copy(rows_v, dst.at[pl.ds(base, BPW)])


def _tc_mlp_body(ue_ref, me_ref, up_ref, mp_ref, g_ref, gf_ref, w1_ref,
                 b1_ref, w2_ref, b2_ref, w3_ref, b3_ref, out_ref):
    ue = ue_ref[...]
    me = me_ref[...]
    lane = lax.broadcasted_iota(jnp.int32, (BT, 128), 1)
    upar = lax.bitwise_and(lax.shift_right_logical(up_ref[...], 11), 1)
    mpar = lax.bitwise_and(lax.shift_right_logical(mp_ref[...], 11), 1)
    uem = jnp.where((lane >= D) == (upar == 1), ue, 0.0)
    mem = jnp.where((lane >= D) == (mpar == 1), me, 0.0)
    g = g_ref[0]
    ohT = (lax.broadcasted_iota(jnp.int32, (32, BT), 0) == g).astype(jnp.float32)
    w1 = w1_ref[...]
    w1u = jnp.concatenate([w1[0:D, :], w1[0:D, :]], axis=0)
    w1m = jnp.concatenate([w1[D:2 * D, :], w1[D:2 * D, :]], axis=0)
    gcon = jnp.dot(gf_ref[...], w1[2 * D:3 * D, :],
                   preferred_element_type=jnp.float32)
    h1 = jnp.dot(uem, w1u, preferred_element_type=jnp.float32)
    h1 += jnp.dot(mem, w1m, preferred_element_type=jnp.float32)
    h1 += lax.dot_general(ohT, gcon, (((0,), (0,)), ((), ())),
                          preferred_element_type=jnp.float32)
    h1 = jnp.maximum(h1 + b1_ref[...], 0.0)
    h2 = jnp.dot(h1, w2_ref[...], preferred_element_type=jnp.float32)
    h2 = jnp.maximum(h2 + b2_ref[...], 0.0)
    out = jnp.sum(h2 * w3_ref[...], axis=1, keepdims=True) + b3_ref[...]
    out_ref[...] = out


_tc_mlp = pl.pallas_call(
    _tc_mlp_body,
    grid=(NB,),
    in_specs=[
        pl.BlockSpec((BT, 128), lambda i: (i, 0)),
        pl.BlockSpec((BT, 128), lambda i: (i, 0)),
        pl.BlockSpec((BT, 1), lambda i: (i, 0)),
        pl.BlockSpec((BT, 1), lambda i: (i, 0)),
        pl.BlockSpec((1, 1, BT), lambda i: (i, 0, 0)),
        pl.BlockSpec((32, D), lambda i: (0, 0)),
        pl.BlockSpec((3 * D, 128), lambda i: (0, 0)),
        pl.BlockSpec((1, 128), lambda i: (0, 0)),
        pl.BlockSpec((128, D), lambda i: (0, 0)),
        pl.BlockSpec((1, D), lambda i: (0, 0)),
        pl.BlockSpec((1, D), lambda i: (0, 0)),
        pl.BlockSpec((1, 1), lambda i: (0, 0)),
    ],
    out_specs=pl.BlockSpec((BT, 1), lambda i: (i, 0)),
    out_shape=jax.ShapeDtypeStruct((B, 1), jnp.float32),
)


def kernel(user, movie, genres, user_factors, movie_factors, genre_factors,
           W1, b1, W2, b2, W3, b3):
    ident = jnp.eye(D, dtype=jnp.float32)
    up = _pack_user(user_factors.T, ident)
    mp = _pack_movie(movie_factors.T, ident)
    user = user.astype(jnp.int32)
    movie = movie.astype(jnp.int32)
    half_rows = TBLK // 2

    def _super(idx):
        blk = lax.shift_right_logical(idx, 12)
        return blk * half_rows + lax.bitwise_and(idx, half_rows - 1)

    usup = _super(user).reshape(NW, IDX_CH, 128)
    msup = _super(movie).reshape(NW, IDX_CH, 128)
    ue, me = _sc_gather(usup, msup, up, mp)
    g3 = genres.astype(jnp.int32).reshape(NB, 1, BT)
    return _tc_mlp(ue, me, user.reshape(B, 1), movie.reshape(B, 1),
                   g3, genre_factors,
                   W1, b1.reshape(1, 128), W2, b2.reshape(1, D),
                   W3.reshape(1, D), b3.reshape(1, 1))


# speedup vs baseline: 1.6290x; 1.2184x over previous
import functools

import jax
import jax.numpy as jnp
from jax import lax
from jax.experimental import pallas as pl
from jax.experimental.pallas import tpu as pltpu
from jax.experimental.pallas import tpu_sc as plsc

B = 16384
D = 64
NC = 2
NS = 16
NW = NC * NS
BPW = B // NW
IDX_CH = BPW // 128

BT = 2048
NB = B // BT

TBLK = 8192

_mesh = plsc.VectorSubcoreMesh(core_axis_name="c", subcore_axis_name="s")


def _pack_body(xt_ref, ident_ref, out_ref):
    x = xt_ref[...]
    ident = ident_ref[...]
    xt = lax.dot_general(x, ident, (((0,), (0,)), ((), ())),
                         preferred_element_type=jnp.float32)
    out_ref[...] = jnp.concatenate(
        [xt[:TBLK // 2, :], xt[TBLK // 2:, :]], axis=1)


def _make_pack(n_rows):
    n_blocks = pl.cdiv(n_rows, TBLK)
    return pl.pallas_call(
        _pack_body,
        grid=(n_blocks,),
        in_specs=[
            pl.BlockSpec((D, TBLK), lambda i: (0, i)),
            pl.BlockSpec((D, D), lambda i: (0, 0)),
        ],
        out_specs=pl.BlockSpec((TBLK // 2, 128), lambda i: (i, 0)),
        out_shape=jax.ShapeDtypeStruct((n_blocks * (TBLK // 2), 128),
                                       jnp.float32),
    )


_pack_user = _make_pack(1000000)
_pack_movie = _make_pack(100000)


@functools.partial(
    pl.kernel,
    mesh=_mesh,
    out_type=[
        jax.ShapeDtypeStruct((B, 128), jnp.float32),
        jax.ShapeDtypeStruct((B, 128), jnp.float32),
    ],
    scratch_types=[
        pltpu.VMEM((IDX_CH, 128), jnp.int32),
        pltpu.VMEM((IDX_CH, 128), jnp.int32),
        pltpu.VMEM((BPW, 128), jnp.float32),
        pltpu.SemaphoreType.DMA,
    ],
)
def _sc_gather(uidx_hbm, midx_hbm, up_hbm, mp_hbm, ue_out, me_out,
               uidx_v, midx_v, rows_v, sem):
    wid = lax.axis_index("s") * NC + lax.axis_index("c")
    base = wid * BPW
    pltpu.sync_copy(uidx_hbm.at[wid], uidx_v)
    pltpu.sync_copy(midx_hbm.at[wid], midx_v)
    for idx_v, src, dst in ((uidx_v, up_hbm, ue_out), (midx_v, mp_hbm, me_out)):
        copies = [
            pltpu.async_copy(
                src.at[idx_v.at[j]], rows_v.at[pl.ds(j * 128, 128)], sem)
            for j in range(IDX_CH)
        ]
        for c in copies:
            c.wait()
        pltpu.sync_copy(rows_v, dst.at[pl.ds(base, BPW)])


def _tc_mlp_body(ue_ref, me_ref, up_ref, mp_ref, g_ref, gf_ref, w1_ref,
                 b1_ref, w2_ref, b2_ref, w3_ref, b3_ref, out_ref):
    ue = ue_ref[...]
    me = me_ref[...]
    lane = lax.broadcasted_iota(jnp.int32, (BT, 128), 1)
    upar = lax.bitwise_and(lax.shift_right_logical(up_ref[...], 12), 1)
    mpar = lax.bitwise_and(lax.shift_right_logical(mp_ref[...], 12), 1)
    uem = jnp.where((lane >= D) == (upar == 1), ue, 0.0)
    mem = jnp.where((lane >= D) == (mpar == 1), me, 0.0)
    g = g_ref[0]
    ohT = (lax.broadcasted_iota(jnp.int32, (32, BT), 0) == g).astype(jnp.float32)
    w1 = w1_ref[...]
    w1u = jnp.concatenate([w1[0:D, :], w1[0:D, :]], axis=0)
    w1m = jnp.concatenate([w1[D:2 * D, :], w1[D:2 * D, :]], axis=0)
    gcon = jnp.dot(gf_ref[...], w1[2 * D:3 * D, :],
                   preferred_element_type=jnp.float32)
    h1 = jnp.dot(uem, w1u, preferred_element_type=jnp.float32)
    h1 += jnp.dot(mem, w1m, preferred_element_type=jnp.float32)
    h1 += lax.dot_general(ohT, gcon, (((0,), (0,)), ((), ())),
                          preferred_element_type=jnp.float32)
    h1 = jnp.maximum(h1 + b1_ref[...], 0.0)
    h2 = jnp.dot(h1, w2_ref[...], preferred_element_type=jnp.float32)
    h2 = jnp.maximum(h2 + b2_ref[...], 0.0)
    out = jnp.sum(h2 * w3_ref[...], axis=1, keepdims=True) + b3_ref[...]
    out_ref[...] = out


_tc_mlp = pl.pallas_call(
    _tc_mlp_body,
    grid=(NB,),
    in_specs=[
        pl.BlockSpec((BT, 128), lambda i: (i, 0)),
        pl.BlockSpec((BT, 128), lambda i: (i, 0)),
        pl.BlockSpec((BT, 1), lambda i: (i, 0)),
        pl.BlockSpec((BT, 1), lambda i: (i, 0)),
        pl.BlockSpec((1, 1, BT), lambda i: (i, 0, 0)),
        pl.BlockSpec((32, D), lambda i: (0, 0)),
        pl.BlockSpec((3 * D, 128), lambda i: (0, 0)),
        pl.BlockSpec((1, 128), lambda i: (0, 0)),
        pl.BlockSpec((128, D), lambda i: (0, 0)),
        pl.BlockSpec((1, D), lambda i: (0, 0)),
        pl.BlockSpec((1, D), lambda i: (0, 0)),
        pl.BlockSpec((1, 1), lambda i: (0, 0)),
    ],
    out_specs=pl.BlockSpec((BT, 1), lambda i: (i, 0)),
    out_shape=jax.ShapeDtypeStruct((B, 1), jnp.float32),
)


def kernel(user, movie, genres, user_factors, movie_factors, genre_factors,
           W1, b1, W2, b2, W3, b3):
    ident = jnp.eye(D, dtype=jnp.float32)
    up = _pack_user(user_factors.T, ident)
    mp = _pack_movie(movie_factors.T, ident)
    user = user.astype(jnp.int32)
    movie = movie.astype(jnp.int32)
    half_rows = TBLK // 2

    def _super(idx):
        blk = lax.shift_right_logical(idx, 13)
        return blk * half_rows + lax.bitwise_and(idx, half_rows - 1)

    usup = _super(user).reshape(NW, IDX_CH, 128)
    msup = _super(movie).reshape(NW, IDX_CH, 128)
    ue, me = _sc_gather(usup, msup, up, mp)
    g3 = genres.astype(jnp.int32).reshape(NB, 1, BT)
    return _tc_mlp(ue, me, user.reshape(B, 1), movie.reshape(B, 1),
                   g3, genre_factors,
                   W1, b1.reshape(1, 128), W2, b2.reshape(1, D),
                   W3.reshape(1, D), b3.reshape(1, 1))


# speedup vs baseline: 1.7917x; 1.0999x over previous
import functools

import jax
import jax.numpy as jnp
from jax import lax
from jax.experimental import pallas as pl
from jax.experimental.pallas import tpu as pltpu
from jax.experimental.pallas import tpu_sc as plsc

B = 16384
D = 64
NC = 2
NS = 16
NW = NC * NS
BPW = B // NW
IDX_CH = BPW // 128

BT = 2048
NB = B // BT

TBLK = 16384

_mesh = plsc.VectorSubcoreMesh(core_axis_name="c", subcore_axis_name="s")


def _pack_body(xt_ref, ident_ref, out_ref):
    x = xt_ref[...]
    ident = ident_ref[...]
    xt = lax.dot_general(x, ident, (((0,), (0,)), ((), ())),
                         preferred_element_type=jnp.float32)
    out_ref[...] = jnp.concatenate(
        [xt[:TBLK // 2, :], xt[TBLK // 2:, :]], axis=1)


def _make_pack(n_rows):
    n_blocks = pl.cdiv(n_rows, TBLK)
    return pl.pallas_call(
        _pack_body,
        grid=(n_blocks,),
        in_specs=[
            pl.BlockSpec((D, TBLK), lambda i: (0, i)),
            pl.BlockSpec((D, D), lambda i: (0, 0)),
        ],
        out_specs=pl.BlockSpec((TBLK // 2, 128), lambda i: (i, 0)),
        out_shape=jax.ShapeDtypeStruct((n_blocks * (TBLK // 2), 128),
                                       jnp.float32),
    )


_pack_user = _make_pack(1000000)
_pack_movie = _make_pack(100000)


@functools.partial(
    pl.kernel,
    mesh=_mesh,
    out_type=[
        jax.ShapeDtypeStruct((B, 128), jnp.float32),
        jax.ShapeDtypeStruct((B, 128), jnp.float32),
    ],
    scratch_types=[
        pltpu.VMEM((IDX_CH, 128), jnp.int32),
        pltpu.VMEM((IDX_CH, 128), jnp.int32),
        pltpu.VMEM((BPW, 128), jnp.float32),
        pltpu.SemaphoreType.DMA,
    ],
)
def _sc_gather(uidx_hbm, midx_hbm, up_hbm, mp_hbm, ue_out, me_out,
               uidx_v, midx_v, rows_v, sem):
    wid = lax.axis_index("s") * NC + lax.axis_index("c")
    base = wid * BPW
    pltpu.sync_copy(uidx_hbm.at[wid], uidx_v)
    pltpu.sync_copy(midx_hbm.at[wid], midx_v)
    for idx_v, src, dst in ((uidx_v, up_hbm, ue_out), (midx_v, mp_hbm, me_out)):
        copies = [
            pltpu.async_copy(
                src.at[idx_v.at[j]], rows_v.at[pl.ds(j * 128, 128)], sem)
            for j in range(IDX_CH)
        ]
        for c in copies:
            c.wait()
        pltpu.sync_copy(rows_v, dst.at[pl.ds(base, BPW)])


def _tc_mlp_body(ue_ref, me_ref, up_ref, mp_ref, g_ref, gf_ref, w1_ref,
                 b1_ref, w2_ref, b2_ref, w3_ref, b3_ref, out_ref):
    ue = ue_ref[...]
    me = me_ref[...]
    lane = lax.broadcasted_iota(jnp.int32, (BT, 128), 1)
    upar = lax.bitwise_and(lax.shift_right_logical(up_ref[...], 13), 1)
    mpar = lax.bitwise_and(lax.shift_right_logical(mp_ref[...], 13), 1)
    uem = jnp.where((lane >= D) == (upar == 1), ue, 0.0)
    mem = jnp.where((lane >= D) == (mpar == 1), me, 0.0)
    g = g_ref[0]
    ohT = (lax.broadcasted_iota(jnp.int32, (32, BT), 0) == g).astype(jnp.float32)
    w1 = w1_ref[...]
    w1u = jnp.concatenate([w1[0:D, :], w1[0:D, :]], axis=0)
    w1m = jnp.concatenate([w1[D:2 * D, :], w1[D:2 * D, :]], axis=0)
    gcon = jnp.dot(gf_ref[...], w1[2 * D:3 * D, :],
                   preferred_element_type=jnp.float32)
    h1 = jnp.dot(uem, w1u, preferred_element_type=jnp.float32)
    h1 += jnp.dot(mem, w1m, preferred_element_type=jnp.float32)
    h1 += lax.dot_general(ohT, gcon, (((0,), (0,)), ((), ())),
                          preferred_element_type=jnp.float32)
    h1 = jnp.maximum(h1 + b1_ref[...], 0.0)
    h2 = jnp.dot(h1, w2_ref[...], preferred_element_type=jnp.float32)
    h2 = jnp.maximum(h2 + b2_ref[...], 0.0)
    out = jnp.sum(h2 * w3_ref[...], axis=1, keepdims=True) + b3_ref[...]
    out_ref[...] = out


_tc_mlp = pl.pallas_call(
    _tc_mlp_body,
    grid=(NB,),
    in_specs=[
        pl.BlockSpec((BT, 128), lambda i: (i, 0)),
        pl.BlockSpec((BT, 128), lambda i: (i, 0)),
        pl.BlockSpec((BT, 1), lambda i: (i, 0)),
        pl.BlockSpec((BT, 1), lambda i: (i, 0)),
        pl.BlockSpec((1, 1, BT), lambda i: (i, 0, 0)),
        pl.BlockSpec((32, D), lambda i: (0, 0)),
        pl.BlockSpec((3 * D, 128), lambda i: (0, 0)),
        pl.BlockSpec((1, 128), lambda i: (0, 0)),
        pl.BlockSpec((128, D), lambda i: (0, 0)),
        pl.BlockSpec((1, D), lambda i: (0, 0)),
        pl.BlockSpec((1, D), lambda i: (0, 0)),
        pl.BlockSpec((1, 1), lambda i: (0, 0)),
    ],
    out_specs=pl.BlockSpec((BT, 1), lambda i: (i, 0)),
    out_shape=jax.ShapeDtypeStruct((B, 1), jnp.float32),
)


def kernel(user, movie, genres, user_factors, movie_factors, genre_factors,
           W1, b1, W2, b2, W3, b3):
    ident = jnp.eye(D, dtype=jnp.float32)
    up = _pack_user(user_factors.T, ident)
    mp = _pack_movie(movie_factors.T, ident)
    user = user.astype(jnp.int32)
    movie = movie.astype(jnp.int32)
    half_rows = TBLK // 2

    def _super(idx):
        blk = lax.shift_right_logical(idx, 14)
        return blk * half_rows + lax.bitwise_and(idx, half_rows - 1)

    usup = _super(user).reshape(NW, IDX_CH, 128)
    msup = _super(movie).reshape(NW, IDX_CH, 128)
    ue, me = _sc_gather(usup, msup, up, mp)
    g3 = genres.astype(jnp.int32).reshape(NB, 1, BT)
    return _tc_mlp(ue, me, user.reshape(B, 1), movie.reshape(B, 1),
                   g3, genre_factors,
                   W1, b1.reshape(1, 128), W2, b2.reshape(1, D),
                   W3.reshape(1, D), b3.reshape(1, 1))


# speedup vs baseline: 1.8517x; 1.0334x over previous
import functools

import jax
import jax.numpy as jnp
from jax import lax
from jax.experimental import pallas as pl
from jax.experimental.pallas import tpu as pltpu
from jax.experimental.pallas import tpu_sc as plsc

B = 16384
D = 64
NC = 2
NS = 16
NW = NC * NS
BPW = B // NW
IDX_CH = BPW // 128

BT = 2048
NB = B // BT

TBLK = 32768

_mesh = plsc.VectorSubcoreMesh(core_axis_name="c", subcore_axis_name="s")


def _pack_body(xt_ref, ident_ref, out_ref):
    x = xt_ref[...]
    ident = ident_ref[...]
    xt = lax.dot_general(x, ident, (((0,), (0,)), ((), ())),
                         preferred_element_type=jnp.float32)
    out_ref[...] = jnp.concatenate(
        [xt[:TBLK // 2, :], xt[TBLK // 2:, :]], axis=1)


def _make_pack(n_rows):
    n_blocks = pl.cdiv(n_rows, TBLK)
    return pl.pallas_call(
        _pack_body,
        grid=(n_blocks,),
        in_specs=[
            pl.BlockSpec((D, TBLK), lambda i: (0, i)),
            pl.BlockSpec((D, D), lambda i: (0, 0)),
        ],
        out_specs=pl.BlockSpec((TBLK // 2, 128), lambda i: (i, 0)),
        out_shape=jax.ShapeDtypeStruct((n_blocks * (TBLK // 2), 128),
                                       jnp.float32),
    )


_pack_user = _make_pack(1000000)
_pack_movie = _make_pack(100000)


@functools.partial(
    pl.kernel,
    mesh=_mesh,
    out_type=[
        jax.ShapeDtypeStruct((B, 128), jnp.float32),
        jax.ShapeDtypeStruct((B, 128), jnp.float32),
    ],
    scratch_types=[
        pltpu.VMEM((IDX_CH, 128), jnp.int32),
        pltpu.VMEM((IDX_CH, 128), jnp.int32),
        pltpu.VMEM((BPW, 128), jnp.float32),
        pltpu.SemaphoreType.DMA,
    ],
)
def _sc_gather(uidx_hbm, midx_hbm, up_hbm, mp_hbm, ue_out, me_out,
               uidx_v, midx_v, rows_v, sem):
    wid = lax.axis_index("s") * NC + lax.axis_index("c")
    base = wid * BPW
    pltpu.sync_copy(uidx_hbm.at[wid], uidx_v)
    pltpu.sync_copy(midx_hbm.at[wid], midx_v)
    for idx_v, src, dst in ((uidx_v, up_hbm, ue_out), (midx_v, mp_hbm, me_out)):
        copies = [
            pltpu.async_copy(
                src.at[idx_v.at[j]], rows_v.at[pl.ds(j * 128, 128)], sem)
            for j in range(IDX_CH)
        ]
        for c in copies:
            c.wait()
        pltpu.sync_copy(rows_v, dst.at[pl.ds(base, BPW)])


def _tc_mlp_body(ue_ref, me_ref, up_ref, mp_ref, g_ref, gf_ref, w1_ref,
                 b1_ref, w2_ref, b2_ref, w3_ref, b3_ref, out_ref):
    ue = ue_ref[...]
    me = me_ref[...]
    lane = lax.broadcasted_iota(jnp.int32, (BT, 128), 1)
    upar = lax.bitwise_and(lax.shift_right_logical(up_ref[...], 14), 1)
    mpar = lax.bitwise_and(lax.shift_right_logical(mp_ref[...], 14), 1)
    uem = jnp.where((lane >= D) == (upar == 1), ue, 0.0)
    mem = jnp.where((lane >= D) == (mpar == 1), me, 0.0)
    g = g_ref[0]
    ohT = (lax.broadcasted_iota(jnp.int32, (32, BT), 0) == g).astype(jnp.float32)
    w1 = w1_ref[...]
    w1u = jnp.concatenate([w1[0:D, :], w1[0:D, :]], axis=0)
    w1m = jnp.concatenate([w1[D:2 * D, :], w1[D:2 * D, :]], axis=0)
    gcon = jnp.dot(gf_ref[...], w1[2 * D:3 * D, :],
                   preferred_element_type=jnp.float32)
    h1 = jnp.dot(uem, w1u, preferred_element_type=jnp.float32)
    h1 += jnp.dot(mem, w1m, preferred_element_type=jnp.float32)
    h1 += lax.dot_general(ohT, gcon, (((0,), (0,)), ((), ())),
                          preferred_element_type=jnp.float32)
    h1 = jnp.maximum(h1 + b1_ref[...], 0.0)
    h2 = jnp.dot(h1, w2_ref[...], preferred_element_type=jnp.float32)
    h2 = jnp.maximum(h2 + b2_ref[...], 0.0)
    out = jnp.sum(h2 * w3_ref[...], axis=1, keepdims=True) + b3_ref[...]
    out_ref[...] = out


_tc_mlp = pl.pallas_call(
    _tc_mlp_body,
    grid=(NB,),
    in_specs=[
        pl.BlockSpec((BT, 128), lambda i: (i, 0)),
        pl.BlockSpec((BT, 128), lambda i: (i, 0)),
        pl.BlockSpec((BT, 1), lambda i: (i, 0)),
        pl.BlockSpec((BT, 1), lambda i: (i, 0)),
        pl.BlockSpec((1, 1, BT), lambda i: (i, 0, 0)),
        pl.BlockSpec((32, D), lambda i: (0, 0)),
        pl.BlockSpec((3 * D, 128), lambda i: (0, 0)),
        pl.BlockSpec((1, 128), lambda i: (0, 0)),
        pl.BlockSpec((128, D), lambda i: (0, 0)),
        pl.BlockSpec((1, D), lambda i: (0, 0)),
        pl.BlockSpec((1, D), lambda i: (0, 0)),
        pl.BlockSpec((1, 1), lambda i: (0, 0)),
    ],
    out_specs=pl.BlockSpec((BT, 1), lambda i: (i, 0)),
    out_shape=jax.ShapeDtypeStruct((B, 1), jnp.float32),
)


def kernel(user, movie, genres, user_factors, movie_factors, genre_factors,
           W1, b1, W2, b2, W3, b3):
    ident = jnp.eye(D, dtype=jnp.float32)
    up = _pack_user(user_factors.T, ident)
    mp = _pack_movie(movie_factors.T, ident)
    user = user.astype(jnp.int32)
    movie = movie.astype(jnp.int32)
    half_rows = TBLK // 2

    def _super(idx):
        blk = lax.shift_right_logical(idx, 15)
        return blk * half_rows + lax.bitwise_and(idx, half_rows - 1)

    usup = _super(user).reshape(NW, IDX_CH, 128)
    msup = _super(movie).reshape(NW, IDX_CH, 128)
    ue, me = _sc_gather(usup, msup, up, mp)
    g3 = genres.astype(jnp.int32).reshape(NB, 1, BT)
    return _tc_mlp(ue, me, user.reshape(B, 1), movie.reshape(B, 1),
                   g3, genre_factors,
                   W1, b1.reshape(1, 128), W2, b2.reshape(1, D),
                   W3.reshape(1, D), b3.reshape(1, 1))


# speedup vs baseline: 1.8775x; 1.0139x over previous
import functools

import jax
import jax.numpy as jnp
from jax import lax
from jax.experimental import pallas as pl
from jax.experimental.pallas import tpu as pltpu
from jax.experimental.pallas import tpu_sc as plsc

B = 16384
D = 64
NC = 2
NS = 16
NW = NC * NS
BPW = B // NW
IDX_CH = BPW // 128

BT = 2048
NB = B // BT

TBLK = 32768

_mesh = plsc.VectorSubcoreMesh(core_axis_name="c", subcore_axis_name="s")


def _pack_body(xt_ref, ident_ref, out_ref):
    x = xt_ref[...]
    ident = ident_ref[...]
    xt = lax.dot_general(x, ident, (((0,), (0,)), ((), ())),
                         preferred_element_type=jnp.float32)
    out_ref[...] = jnp.concatenate(
        [xt[:TBLK // 2, :], xt[TBLK // 2:, :]], axis=1)


def _make_pack(n_rows):
    n_blocks = pl.cdiv(n_rows, TBLK)
    return pl.pallas_call(
        _pack_body,
        grid=(n_blocks,),
        in_specs=[
            pl.BlockSpec((D, TBLK), lambda i: (0, i)),
            pl.BlockSpec((D, D), lambda i: (0, 0)),
        ],
        out_specs=pl.BlockSpec((TBLK // 2, 128), lambda i: (i, 0)),
        out_shape=jax.ShapeDtypeStruct((n_blocks * (TBLK // 2), 128),
                                       jnp.float32),
    )


_pack_user = _make_pack(1000000)
_pack_movie = _make_pack(100000)


@functools.partial(
    pl.kernel,
    mesh=_mesh,
    out_type=jax.ShapeDtypeStruct((B, 128), jnp.float32),
    scratch_types=[
        pltpu.VMEM((IDX_CH, 128), jnp.int32),
        pltpu.VMEM((BPW, 128), jnp.float32),
        pltpu.SemaphoreType.DMA,
    ],
)
def _sc_gather(idx_hbm, tab_hbm, out_hbm, idx_v, rows_v, sem):
    wid = lax.axis_index("s") * NC + lax.axis_index("c")
    base = wid * BPW
    pltpu.sync_copy(idx_hbm.at[wid], idx_v)
    copies = [
        pltpu.async_copy(
            tab_hbm.at[idx_v.at[j]], rows_v.at[pl.ds(j * 128, 128)], sem)
        for j in range(IDX_CH)
    ]
    for c in copies:
        c.wait()
    pltpu.sync_copy(rows_v, out_hbm.at[pl.ds(base, BPW)])


def _tc_mlp_body(ue_ref, me_ref, up_ref, mp_ref, g_ref, gf_ref, w1_ref,
                 b1_ref, w2_ref, b2_ref, w3_ref, b3_ref, out_ref):
    ue = ue_ref[...]
    me = me_ref[...]
    lane = lax.broadcasted_iota(jnp.int32, (BT, 128), 1)
    upar = lax.bitwise_and(lax.shift_right_logical(up_ref[...], 14), 1)
    mpar = lax.bitwise_and(lax.shift_right_logical(mp_ref[...], 14), 1)
    uem = jnp.where((lane >= D) == (upar == 1), ue, 0.0)
    mem = jnp.where((lane >= D) == (mpar == 1), me, 0.0)
    g = g_ref[0]
    ohT = (lax.broadcasted_iota(jnp.int32, (32, BT), 0) == g).astype(jnp.float32)
    w1 = w1_ref[...]
    w1u = jnp.concatenate([w1[0:D, :], w1[0:D, :]], axis=0)
    w1m = jnp.concatenate([w1[D:2 * D, :], w1[D:2 * D, :]], axis=0)
    gcon = jnp.dot(gf_ref[...], w1[2 * D:3 * D, :],
                   preferred_element_type=jnp.float32)
    h1 = jnp.dot(uem, w1u, preferred_element_type=jnp.float32)
    h1 += jnp.dot(mem, w1m, preferred_element_type=jnp.float32)
    h1 += lax.dot_general(ohT, gcon, (((0,), (0,)), ((), ())),
                          preferred_element_type=jnp.float32)
    h1 = jnp.maximum(h1 + b1_ref[...], 0.0)
    h2 = jnp.dot(h1, w2_ref[...], preferred_element_type=jnp.float32)
    h2 = jnp.maximum(h2 + b2_ref[...], 0.0)
    out = jnp.sum(h2 * w3_ref[...], axis=1, keepdims=True) + b3_ref[...]
    out_ref[...] = out


_tc_mlp = pl.pallas_call(
    _tc_mlp_body,
    grid=(NB,),
    in_specs=[
        pl.BlockSpec((BT, 128), lambda i: (i, 0)),
        pl.BlockSpec((BT, 128), lambda i: (i, 0)),
        pl.BlockSpec((BT, 1), lambda i: (i, 0)),
        pl.BlockSpec((BT, 1), lambda i: (i, 0)),
        pl.BlockSpec((1, 1, BT), lambda i: (i, 0, 0)),
        pl.BlockSpec((32, D), lambda i: (0, 0)),
        pl.BlockSpec((3 * D, 128), lambda i: (0, 0)),
        pl.BlockSpec((1, 128), lambda i: (0, 0)),
        pl.BlockSpec((128, D), lambda i: (0, 0)),
        pl.BlockSpec((1, D), lambda i: (0, 0)),
        pl.BlockSpec((1, D), lambda i: (0, 0)),
        pl.BlockSpec((1, 1), lambda i: (0, 0)),
    ],
    out_specs=pl.BlockSpec((BT, 1), lambda i: (i, 0)),
    out_shape=jax.ShapeDtypeStruct((B, 1), jnp.float32),
)


def kernel(user, movie, genres, user_factors, movie_factors, genre_factors,
           W1, b1, W2, b2, W3, b3):
    ident = jnp.eye(D, dtype=jnp.float32)
    user = user.astype(jnp.int32)
    movie = movie.astype(jnp.int32)
    half_rows = TBLK // 2

    def _super(idx):
        blk = lax.shift_right_logical(idx, 15)
        return blk * half_rows + lax.bitwise_and(idx, half_rows - 1)

    usup = _super(user).reshape(NW, IDX_CH, 128)
    msup = _super(movie).reshape(NW, IDX_CH, 128)
    mp = _pack_movie(movie_factors.T, ident)
    me = _sc_gather(msup, mp)
    up = _pack_user(user_factors.T, ident)
    ue = _sc_gather(usup, up)
    g3 = genres.astype(jnp.int32).reshape(NB, 1, BT)
    return _tc_mlp(ue, me, user.reshape(B, 1), movie.reshape(B, 1),
                   g3, genre_factors,
                   W1, b1.reshape(1, 128), W2, b2.reshape(1, D),
                   W3.reshape(1, D), b3.reshape(1, 1))
